# baseline (speedup 1.0000x reference)
import jax
import jax.numpy as jnp
from jax import lax
from jax.experimental import pallas as pl
from jax.experimental.pallas import tpu as pltpu
from jax.experimental.pallas import tpu_sc as plsc

NCS = 2
NS = 16
LANES = 16
K_EDGES = 64
G_CHUNKS = 6


def _round_up(a, b):
  return (a + b - 1) // b * b


def _tc_in_body(x_ref, w_ref, asr_ref, adr_ref, h_ref, asad_ref):
  h = jnp.dot(x_ref[...], w_ref[...], preferred_element_type=jnp.float32)
  h_ref[...] = h
  asad_ref[0, :] = jnp.sum(h * asr_ref[...], axis=1)
  asad_ref[1, :] = jnp.sum(h * adr_ref[...], axis=1)


def _tc_mid_body(acc_ref, den_ref, b_ref, w_ref, asr_ref, adr_ref,
                 h_ref, asad_ref):
  d = jnp.sum(den_ref[...], axis=0)
  agg = acc_ref[0] + acc_ref[1]
  hprev = agg / (d[:, None] + 1e-16) + b_ref[...]
  hprev = jnp.maximum(hprev, 0.0)
  h = jnp.dot(hprev, w_ref[...], preferred_element_type=jnp.float32)
  h_ref[...] = h
  asad_ref[0, :] = jnp.sum(h * asr_ref[...], axis=1)
  asad_ref[1, :] = jnp.sum(h * adr_ref[...], axis=1)


def _tc_head_body(acc_ref, den_ref, b_ref, wo_ref, bo_ref, o_ref):
  d = jnp.sum(den_ref[...], axis=0)
  agg = acc_ref[0] + acc_ref[1]
  hprev = agg / (d[:, None] + 1e-16) + b_ref[...]
  hprev = jnp.maximum(hprev, 0.0)
  logits = jnp.dot(hprev, wo_ref[...], preferred_element_type=jnp.float32)
  logits = logits + bo_ref[...]
  m = jnp.max(logits, axis=1, keepdims=True)
  p = jnp.exp(logits - m)
  o_ref[...] = p / jnp.sum(p, axis=1, keepdims=True)


def _sc_edge_call(n_pad, e_pad, d, h, sd, asad):
  stripe = n_pad // NS
  cpw = e_pad // (NCS * NS * K_EDGES)
  assert cpw % G_CHUNKS == 0 and G_CHUNKS % 2 == 0
  ngroups = cpw // G_CHUNKS
  n16 = n_pad // LANES

  def body(h_hbm, sd_hbm, asad_hbm, acc_hbm, den_hbm,
           asrc_v, adst_v, den_v, sd_g, ex_v, rows_a, rows_b,
           acc_s, sem_ga, sem_gb, sem_sa, sem_sb):
    c = lax.axis_index("c")
    s = lax.axis_index("s")
    wid = c * NS + s

    pltpu.sync_copy(asad_hbm.at[0], asrc_v)
    pltpu.sync_copy(asad_hbm.at[1], adst_v)

    zero16 = jnp.zeros((LANES,), jnp.float32)

    def zrow(i, carry):
      for r8 in range(d // LANES):
        rows_a[i, pl.ds(r8 * LANES, LANES)] = zero16
      return carry
    lax.fori_loop(0, K_EDGES, zrow, 0)

    def zden(i, carry):
      den_v[pl.ds(i * LANES, LANES)] = zero16
      return carry
    lax.fori_loop(0, n16, zden, 0)

    off = 0
    while off < stripe:
      step = min(K_EDGES, stripe - off)
      pltpu.sync_copy(rows_a.at[pl.ds(0, step)],
                      acc_s.at[pl.ds(s * stripe + off, step)])
      off += step
    plsc.subcore_barrier()

    row0 = wid * cpw

    def do_chunk(sd_v, rows_b_, g_sem, s_sem):
      for cc in range(K_EDGES // LANES):
        si = sd_v[0, pl.ds(cc * LANES, LANES)]
        di = sd_v[1, pl.ds(cc * LANES, LANES)]
        t = (plsc.load_gather(asrc_v, [si])
             + plsc.load_gather(adst_v, [di]))
        ex = jnp.exp(jnp.maximum(t, t * 0.2))
        ex_v[pl.ds(cc * LANES, LANES)] = ex
        plsc.addupdate_scatter(den_v, [di], ex)
      pltpu.make_async_copy(h_hbm.at[sd_v.at[0]], rows_b_, g_sem).wait()

      def scale(g, carry2):
        exv = ex_v[pl.ds(g * LANES, LANES)]
        for l in range(LANES):
          xv = exv[l]
          row = g * LANES + l
          for r8 in range(d // LANES):
            sl = pl.ds(r8 * LANES, LANES)
            rows_b_[row, sl] = rows_b_[row, sl] * xv
        return carry2
      lax.fori_loop(0, K_EDGES // LANES, scale, 0)

      pltpu.async_copy(rows_b_, acc_s.at[sd_v.at[1]], s_sem, add=True)

    pltpu.sync_copy(sd_hbm.at[pl.ds(row0, G_CHUNKS)], sd_g)
    pltpu.async_copy(h_hbm.at[sd_g.at[0].at[0]], rows_a, sem_ga)
    pltpu.async_copy(h_hbm.at[sd_g.at[1].at[0]], rows_b, sem_gb)

    def group(gi, carry):
      for j in range(0, G_CHUNKS - 2, 2):
        do_chunk(sd_g.at[j], rows_a, sem_ga, sem_sa)
        do_chunk(sd_g.at[j + 1], rows_b, sem_gb, sem_sb)
        pltpu.make_async_copy(rows_a, acc_s.at[sd_g.at[j].at[1]],
                              sem_sa).wait()
        pltpu.async_copy(h_hbm.at[sd_g.at[j + 2].at[0]], rows_a, sem_ga)
        pltpu.make_async_copy(rows_b, acc_s.at[sd_g.at[j + 1].at[1]],
                              sem_sb).wait()
        pltpu.async_copy(h_hbm.at[sd_g.at[j + 3].at[0]], rows_b, sem_gb)
      do_chunk(sd_g.at[G_CHUNKS - 2], rows_a, sem_ga, sem_sa)
      do_chunk(sd_g.at[G_CHUNKS - 1], rows_b, sem_gb, sem_sb)
      pltpu.make_async_copy(rows_a, acc_s.at[sd_g.at[G_CHUNKS - 2].at[1]],
                            sem_sa).wait()
      pltpu.make_async_copy(rows_b, acc_s.at[sd_g.at[G_CHUNKS - 1].at[1]],
                            sem_sb).wait()
      grow = row0 + (gi + 1) * G_CHUNKS
      pltpu.sync_copy(sd_hbm.at[pl.ds(grow, G_CHUNKS)], sd_g)
      pltpu.async_copy(h_hbm.at[sd_g.at[0].at[0]], rows_a, sem_ga)
      pltpu.async_copy(h_hbm.at[sd_g.at[1].at[0]], rows_b, sem_gb)
      return carry
    lax.fori_loop(0, ngroups, group, 0)

    pltpu.make_async_copy(h_hbm.at[sd_g.at[0].at[0]], rows_a,
                          sem_ga).wait()
    pltpu.make_async_copy(h_hbm.at[sd_g.at[1].at[0]], rows_b,
                          sem_gb).wait()
    plsc.subcore_barrier()

    pltpu.sync_copy(den_v, den_hbm.at[wid])
    pltpu.sync_copy(acc_s.at[pl.ds(s * stripe, stripe)],
                    acc_hbm.at[c].at[pl.ds(s * stripe, stripe)])

  mesh = plsc.VectorSubcoreMesh(core_axis_name="c", subcore_axis_name="s",
                                num_cores=NCS, num_subcores=NS)
  fn = pl.kernel(
      body,
      out_type=[jax.ShapeDtypeStruct((NCS, n_pad, d), jnp.float32),
                jax.ShapeDtypeStruct((NCS * NS, n_pad), jnp.float32)],
      mesh=mesh,
      scratch_types=[
          pltpu.VMEM((n_pad,), jnp.float32),
          pltpu.VMEM((n_pad,), jnp.float32),
          pltpu.VMEM((n_pad,), jnp.float32),
          pltpu.VMEM((G_CHUNKS, 2, K_EDGES), jnp.int32),
          pltpu.VMEM((K_EDGES,), jnp.float32),
          pltpu.VMEM((K_EDGES, d), jnp.float32),
          pltpu.VMEM((K_EDGES, d), jnp.float32),
          pltpu.VMEM_SHARED((n_pad, d), jnp.float32),
          pltpu.SemaphoreType.DMA,
          pltpu.SemaphoreType.DMA,
          pltpu.SemaphoreType.DMA,
          pltpu.SemaphoreType.DMA,
      ],
      compiler_params=pltpu.CompilerParams(needs_layout_passes=False),
  )
  return fn(h, sd, asad)


def kernel(x, edge_index, W1, a_src1, a_dst1, b1, W2, a_src2, a_dst2, b2,
           Wo, bo):
  n, d_in = x.shape
  d_h = W1.shape[1]
  n_classes = Wo.shape[1]
  e = edge_index.shape[1]
  n_pad = _round_up(n + 1, 128)
  e_pad = _round_up(e + n, NCS * NS * K_EDGES * G_CHUNKS)
  e_tot = e_pad + G_CHUNKS * K_EDGES

  ei = edge_index.astype(jnp.int32)
  loop = jnp.arange(n, dtype=jnp.int32)
  pad_e = e_tot - e - n
  srcf = jnp.concatenate(
      [ei[0], loop, jnp.zeros((pad_e,), jnp.int32)]).reshape(-1, K_EDGES)
  dstf = jnp.concatenate(
      [ei[1], loop, jnp.full((pad_e,), n, jnp.int32)]).reshape(-1, K_EDGES)
  sd = jnp.stack([srcf, dstf], axis=1)

  x_pad = jnp.pad(x, ((0, n_pad - n), (0, 0)))

  vec = lambda a: a.reshape(1, -1)

  h1, asad1 = pl.pallas_call(
      _tc_in_body,
      out_shape=[jax.ShapeDtypeStruct((n_pad, d_h), jnp.float32),
                 jax.ShapeDtypeStruct((2, n_pad), jnp.float32)],
  )(x_pad, W1, vec(a_src1), vec(a_dst1))

  acc1, den1 = _sc_edge_call(n_pad, e_pad, d_h, h1, sd, asad1)

  h2, asad2 = pl.pallas_call(
      _tc_mid_body,
      out_shape=[jax.ShapeDtypeStruct((n_pad, d_h), jnp.float32),
                 jax.ShapeDtypeStruct((2, n_pad), jnp.float32)],
  )(acc1, den1, vec(b1), W2, vec(a_src2), vec(a_dst2))

  acc2, den2 = _sc_edge_call(n_pad, e_pad, d_h, h2, sd, asad2)

  out = pl.pallas_call(
      _tc_head_body,
      out_shape=jax.ShapeDtypeStruct((n_pad, n_classes), jnp.float32),
  )(acc2, den2, vec(b2), Wo, vec(bo))

  return out[:n]


# speedup vs baseline: 1.1487x; 1.1487x over previous
import jax
import jax.numpy as jnp
from jax import lax
from jax.experimental import pallas as pl
from jax.experimental.pallas import tpu as pltpu
from jax.experimental.pallas import tpu_sc as plsc

NCS = 2
NS = 16
LANES = 16
K_EDGES = 64
G_CHUNKS = 9


def _round_up(a, b):
  return (a + b - 1) // b * b


def _pack_logits(as_, ad_):
  au = jax.lax.bitcast_convert_type(as_, jnp.uint32)
  au = (au + 0x8000) & jnp.uint32(0xFFFF0000)
  du = jax.lax.bitcast_convert_type(ad_, jnp.uint32)
  du = (du + 0x8000) >> 16
  return jax.lax.bitcast_convert_type(au | du, jnp.float32)


def _tc_in_body(x_ref, w_ref, asr_ref, adr_ref, h_ref, asad_ref):
  h = jnp.dot(x_ref[...], w_ref[...], preferred_element_type=jnp.float32)
  h_ref[...] = h
  asad_ref[0, :] = _pack_logits(jnp.sum(h * asr_ref[...], axis=1),
                                jnp.sum(h * adr_ref[...], axis=1))


def _tc_mid_body(acc_ref, den_ref, b_ref, w_ref, asr_ref, adr_ref,
                 h_ref, asad_ref):
  d = jnp.sum(den_ref[...], axis=0)
  agg = acc_ref[0] + acc_ref[1]
  hprev = agg / (d[:, None] + 1e-16) + b_ref[...]
  hprev = jnp.maximum(hprev, 0.0)
  h = jnp.dot(hprev, w_ref[...], preferred_element_type=jnp.float32)
  h_ref[...] = h
  asad_ref[0, :] = _pack_logits(jnp.sum(h * asr_ref[...], axis=1),
                                jnp.sum(h * adr_ref[...], axis=1))


def _tc_head_body(acc_ref, den_ref, b_ref, wo_ref, bo_ref, o_ref):
  d = jnp.sum(den_ref[...], axis=0)
  agg = acc_ref[0] + acc_ref[1]
  hprev = agg / (d[:, None] + 1e-16) + b_ref[...]
  hprev = jnp.maximum(hprev, 0.0)
  logits = jnp.dot(hprev, wo_ref[...], preferred_element_type=jnp.float32)
  logits = logits + bo_ref[...]
  m = jnp.max(logits, axis=1, keepdims=True)
  p = jnp.exp(logits - m)
  o_ref[...] = p / jnp.sum(p, axis=1, keepdims=True)


def _sc_edge_call(n_pad, e_pad, d, h, sd, asad):
  stripe = n_pad // NS
  cpw = e_pad // (NCS * NS * K_EDGES)
  assert cpw % G_CHUNKS == 0 and G_CHUNKS % 3 == 0
  ngroups = cpw // G_CHUNKS
  n16 = n_pad // LANES
  row_bytes = K_EDGES * d * 4
  G = G_CHUNKS

  def body(h_hbm, sd_hbm, asad_hbm, acc_hbm, den_hbm,
           pk_v, den_v, sd_g, ex_v, rows_a, rows_b, rows_c,
           acc_s, sem_g0, sem_g1, sem_g2, sem_s0, sem_s1, sem_s2):
    c = lax.axis_index("c")
    s = lax.axis_index("s")
    wid = c * NS + s
    rows = [rows_a, rows_b, rows_c]
    gsem = [sem_g0, sem_g1, sem_g2]
    ssem = [sem_s0, sem_s1, sem_s2]

    pltpu.sync_copy(asad_hbm.at[0], pk_v)

    zero16 = jnp.zeros((LANES,), jnp.float32)

    def zrow(i, carry):
      for r8 in range(d // LANES):
        rows_a[i, pl.ds(r8 * LANES, LANES)] = zero16
        rows_c[i, pl.ds(r8 * LANES, LANES)] = zero16
      return carry
    lax.fori_loop(0, K_EDGES, zrow, 0)

    def zden(i, carry):
      den_v[pl.ds(i * LANES, LANES)] = zero16
      return carry
    lax.fori_loop(0, n16, zden, 0)

    off = 0
    while off < stripe:
      step = min(K_EDGES, stripe - off)
      pltpu.sync_copy(rows_a.at[pl.ds(0, step)],
                      acc_s.at[pl.ds(s * stripe + off, step)])
      off += step
    plsc.subcore_barrier()

    row0 = wid * cpw
    himask = jnp.full((LANES,), 0xFFFF0000, jnp.uint32)

    def do_ex(sd_v):
      for cc in range(K_EDGES // LANES):
        si = sd_v[0, pl.ds(cc * LANES, LANES)]
        di = sd_v[1, pl.ds(cc * LANES, LANES)]
        pks = plsc.bitcast(plsc.load_gather(pk_v, [si]), jnp.uint32)
        pkd = plsc.bitcast(plsc.load_gather(pk_v, [di]), jnp.uint32)
        t = (plsc.bitcast(pks & himask, jnp.float32)
             + plsc.bitcast(pkd << 16, jnp.float32))
        ex = jnp.exp(jnp.maximum(t, t * 0.2))
        ex_v[pl.ds(cc * LANES, LANES)] = ex
        plsc.addupdate_scatter(den_v, [di], ex)

    def do_scale(rows_b_):
      def scale(g, carry2):
        exv = ex_v[pl.ds(g * LANES, LANES)]
        for l in range(LANES):
          xv = exv[l]
          row = g * LANES + l
          for r8 in range(d // LANES):
            sl = pl.ds(r8 * LANES, LANES)
            rows_b_[row, sl] = rows_b_[row, sl] * xv
        return carry2
      lax.fori_loop(0, K_EDGES // LANES, scale, 0)

    pltpu.sync_copy(sd_hbm.at[pl.ds(row0, G + 2)], sd_g)
    pltpu.async_copy(h_hbm.at[sd_g.at[0].at[0]], rows_a, sem_g0)
    pltpu.async_copy(h_hbm.at[sd_g.at[1].at[0]], rows_b, sem_g1)
    pltpu.make_async_copy(h_hbm.at[sd_g.at[0].at[0]], rows_a,
                          sem_g0).wait()
    pltpu.make_async_copy(h_hbm.at[sd_g.at[1].at[0]], rows_b,
                          sem_g1).wait()
    pltpu.async_copy(rows_c, acc_s.at[sd_g.at[0].at[1]], sem_s2, add=True)

    def group(gi, carry):
      for jj in range(G):
        bp = (jj - 1) % 3
        bc = jj % 3
        bn = (jj + 2) % 3
        pltpu.make_async_copy(rows[bp], acc_s.at[sd_g.at[max(jj - 1, 0)]
                                                 .at[1]], ssem[bp]).wait()
        pltpu.async_copy(h_hbm.at[sd_g.at[jj + 2].at[0]], rows[bn],
                         gsem[bn])
        do_ex(sd_g.at[jj])
        if jj >= 2:
          pltpu.make_async_copy(h_hbm.at[sd_g.at[jj].at[0]], rows[bc],
                                gsem[bc]).wait()
        do_scale(rows[bc])
        pltpu.async_copy(rows[bc], acc_s.at[sd_g.at[jj].at[1]], ssem[bc],
                         add=True)
      pltpu.make_async_copy(h_hbm.at[sd_g.at[G].at[0]], rows_a,
                            sem_g0).wait()
      pltpu.make_async_copy(h_hbm.at[sd_g.at[G + 1].at[0]], rows_b,
                            sem_g1).wait()
      grow = row0 + (gi + 1) * G
      pltpu.sync_copy(sd_hbm.at[pl.ds(grow, G + 2)], sd_g)
      return carry
    lax.fori_loop(0, ngroups, group, 0)

    pltpu.make_async_copy(rows_c, acc_s.at[sd_g.at[0].at[1]],
                          sem_s2).wait()
    plsc.subcore_barrier()

    pltpu.sync_copy(den_v, den_hbm.at[wid])
    pltpu.sync_copy(acc_s.at[pl.ds(s * stripe, stripe)],
                    acc_hbm.at[c].at[pl.ds(s * stripe, stripe)])

  mesh = plsc.VectorSubcoreMesh(core_axis_name="c", subcore_axis_name="s",
                                num_cores=NCS, num_subcores=NS)
  fn = pl.kernel(
      body,
      out_type=[jax.ShapeDtypeStruct((NCS, n_pad, d), jnp.float32),
                jax.ShapeDtypeStruct((NCS * NS, n_pad), jnp.float32)],
      mesh=mesh,
      scratch_types=[
          pltpu.VMEM((n_pad,), jnp.float32),
          pltpu.VMEM((n_pad,), jnp.float32),
          pltpu.VMEM((G_CHUNKS + 2, 2, K_EDGES), jnp.int32),
          pltpu.VMEM((K_EDGES,), jnp.float32),
          pltpu.VMEM((K_EDGES, d), jnp.float32),
          pltpu.VMEM((K_EDGES, d), jnp.float32),
          pltpu.VMEM((K_EDGES, d), jnp.float32),
          pltpu.VMEM_SHARED((n_pad, d), jnp.float32),
          pltpu.SemaphoreType.DMA,
          pltpu.SemaphoreType.DMA,
          pltpu.SemaphoreType.DMA,
          pltpu.SemaphoreType.DMA,
          pltpu.SemaphoreType.DMA,
          pltpu.SemaphoreType.DMA,
      ],
      compiler_params=pltpu.CompilerParams(needs_layout_passes=False),
  )
  return fn(h, sd, asad)


def kernel(x, edge_index, W1, a_src1, a_dst1, b1, W2, a_src2, a_dst2, b2,
           Wo, bo):
  n, d_in = x.shape
  d_h = W1.shape[1]
  n_classes = Wo.shape[1]
  e = edge_index.shape[1]
  n_pad = _round_up(n + 1, 128)
  e_pad = _round_up(e + n, NCS * NS * K_EDGES * G_CHUNKS)
  e_tot = e_pad + (G_CHUNKS + 2) * K_EDGES

  ei = edge_index.astype(jnp.int32)
  loop = jnp.arange(n, dtype=jnp.int32)
  pad_e = e_tot - e - n
  srcf = jnp.concatenate(
      [ei[0], loop, jnp.zeros((pad_e,), jnp.int32)]).reshape(-1, K_EDGES)
  dstf = jnp.concatenate(
      [ei[1], loop, jnp.full((pad_e,), n, jnp.int32)]).reshape(-1, K_EDGES)
  sd = jnp.stack([srcf, dstf], axis=1)

  x_pad = jnp.pad(x, ((0, n_pad - n), (0, 0)))

  vec = lambda a: a.reshape(1, -1)

  h1, asad1 = pl.pallas_call(
      _tc_in_body,
      out_shape=[jax.ShapeDtypeStruct((n_pad, d_h), jnp.float32),
                 jax.ShapeDtypeStruct((1, n_pad), jnp.float32)],
  )(x_pad, W1, vec(a_src1), vec(a_dst1))

  acc1, den1 = _sc_edge_call(n_pad, e_pad, d_h, h1, sd, asad1)

  h2, asad2 = pl.pallas_call(
      _tc_mid_body,
      out_shape=[jax.ShapeDtypeStruct((n_pad, d_h), jnp.float32),
                 jax.ShapeDtypeStruct((1, n_pad), jnp.float32)],
  )(acc1, den1, vec(b1), W2, vec(a_src2), vec(a_dst2))

  acc2, den2 = _sc_edge_call(n_pad, e_pad, d_h, h2, sd, asad2)

  out = pl.pallas_call(
      _tc_head_body,
      out_shape=jax.ShapeDtypeStruct((n_pad, n_classes), jnp.float32),
  )(acc2, den2, vec(b2), Wo, vec(bo))

  return out[:n]


# speedup vs baseline: 1.1500x; 1.0012x over previous
import jax
import jax.numpy as jnp
from jax import lax
from jax.experimental import pallas as pl
from jax.experimental.pallas import tpu as pltpu
from jax.experimental.pallas import tpu_sc as plsc

NCS = 2
NS = 16
LANES = 16
K_EDGES = 64
G_CHUNKS = 9


def _round_up(a, b):
  return (a + b - 1) // b * b


def _pack_logits(as_, ad_):
  au = jax.lax.bitcast_convert_type(as_, jnp.uint32)
  au = (au + 0x8000) & jnp.uint32(0xFFFF0000)
  du = jax.lax.bitcast_convert_type(ad_, jnp.uint32)
  du = (du + 0x8000) >> 16
  return jax.lax.bitcast_convert_type(au | du, jnp.float32)


def _tc_in_body(x_ref, w_ref, asr_ref, adr_ref, h_ref, asad_ref):
  h = jnp.dot(x_ref[...], w_ref[...], preferred_element_type=jnp.float32)
  h_ref[...] = h
  asad_ref[0, :] = _pack_logits(jnp.sum(h * asr_ref[...], axis=1),
                                jnp.sum(h * adr_ref[...], axis=1))


def _tc_mid_body(acc_ref, den_ref, b_ref, w_ref, asr_ref, adr_ref,
                 h_ref, asad_ref):
  d = jnp.sum(den_ref[...], axis=0)
  agg = acc_ref[0] + acc_ref[1]
  hprev = agg / (d[:, None] + 1e-16) + b_ref[...]
  hprev = jnp.maximum(hprev, 0.0)
  h = jnp.dot(hprev, w_ref[...], preferred_element_type=jnp.float32)
  h_ref[...] = h
  asad_ref[0, :] = _pack_logits(jnp.sum(h * asr_ref[...], axis=1),
                                jnp.sum(h * adr_ref[...], axis=1))


def _tc_head_body(acc_ref, den_ref, b_ref, wo_ref, bo_ref, o_ref):
  d = jnp.sum(den_ref[...], axis=0)
  agg = acc_ref[0] + acc_ref[1]
  hprev = agg / (d[:, None] + 1e-16) + b_ref[...]
  hprev = jnp.maximum(hprev, 0.0)
  logits = jnp.dot(hprev, wo_ref[...], preferred_element_type=jnp.float32)
  logits = logits + bo_ref[...]
  m = jnp.max(logits, axis=1, keepdims=True)
  p = jnp.exp(logits - m)
  o_ref[...] = p / jnp.sum(p, axis=1, keepdims=True)


def _sc_edge_call(n_pad, e_pad, d, h, sd, asad):
  stripe = n_pad // NS
  cpw = e_pad // (NCS * NS * K_EDGES)
  assert cpw % G_CHUNKS == 0 and G_CHUNKS % 3 == 0
  ngroups = cpw // G_CHUNKS
  n16 = n_pad // LANES
  G = G_CHUNKS

  def body(h_hbm, sd_hbm, asad_hbm, acc_hbm, den_hbm,
           pk_v, den_v, sd_g, ex_v, rows_a, rows_b, rows_c,
           acc_s, sem_g0, sem_g1, sem_g2, sem_s0, sem_s1, sem_s2):
    c = lax.axis_index("c")
    s = lax.axis_index("s")
    wid = c * NS + s
    rows = [rows_a, rows_b, rows_c]
    gsem = [sem_g0, sem_g1, sem_g2]
    ssem = [sem_s0, sem_s1, sem_s2]

    pltpu.sync_copy(asad_hbm.at[0], pk_v)

    zero16 = jnp.zeros((LANES,), jnp.float32)

    def zrow(i, carry):
      for r8 in range(d // LANES):
        rows_a[i, pl.ds(r8 * LANES, LANES)] = zero16
        rows_c[i, pl.ds(r8 * LANES, LANES)] = zero16
      return carry
    lax.fori_loop(0, K_EDGES, zrow, 0)

    def zden(i, carry):
      den_v[pl.ds(i * LANES, LANES)] = zero16
      return carry
    lax.fori_loop(0, n16, zden, 0)

    off = 0
    while off < stripe:
      step = min(K_EDGES, stripe - off)
      pltpu.sync_copy(rows_a.at[pl.ds(0, step)],
                      acc_s.at[pl.ds(s * stripe + off, step)])
      off += step
    plsc.subcore_barrier()

    row0 = wid * cpw
    himask = jnp.full((LANES,), 0xFFFF0000, jnp.uint32)

    def do_ex(sd_v):
      for cc in range(K_EDGES // LANES):
        si = sd_v[0, pl.ds(cc * LANES, LANES)]
        di = sd_v[1, pl.ds(cc * LANES, LANES)]
        pks = plsc.bitcast(plsc.load_gather(pk_v, [si]), jnp.uint32)
        pkd = plsc.bitcast(plsc.load_gather(pk_v, [di]), jnp.uint32)
        t = (plsc.bitcast(pks & himask, jnp.float32)
             + plsc.bitcast(pkd << 16, jnp.float32))
        ex = jnp.exp(jnp.maximum(t, t * 0.2))
        ex_v[pl.ds(cc * LANES, LANES)] = ex
        plsc.addupdate_scatter(den_v, [di], ex)

    def do_scale(rows_b_):
      def scale(g, carry2):
        exv = ex_v[pl.ds(g * LANES, LANES)]
        for l in range(LANES):
          xv = exv[l]
          row = g * LANES + l
          for r8 in range(d // LANES):
            sl = pl.ds(r8 * LANES, LANES)
            rows_b_[row, sl] = rows_b_[row, sl] * xv
        return carry2
      lax.fori_loop(0, K_EDGES // LANES, scale, 0)

    pltpu.sync_copy(sd_hbm.at[pl.ds(row0, G + 2)], sd_g)
    pltpu.async_copy(h_hbm.at[sd_g.at[0].at[0]], rows_a, sem_g0)
    pltpu.async_copy(h_hbm.at[sd_g.at[1].at[0]], rows_b, sem_g1)
    pltpu.make_async_copy(h_hbm.at[sd_g.at[0].at[0]], rows_a,
                          sem_g0).wait()
    pltpu.make_async_copy(h_hbm.at[sd_g.at[1].at[0]], rows_b,
                          sem_g1).wait()
    pltpu.async_copy(rows_c, acc_s.at[sd_g.at[0].at[1]], sem_s2, add=True)

    def group(gi, carry):
      for jj in range(G):
        bp = (jj - 1) % 3
        bc = jj % 3
        bn = (jj + 2) % 3
        pltpu.make_async_copy(rows[bp], acc_s.at[sd_g.at[max(jj - 1, 0)]
                                                 .at[1]], ssem[bp]).wait()
        pltpu.async_copy(h_hbm.at[sd_g.at[jj + 2].at[0]], rows[bn],
                         gsem[bn])
        do_ex(sd_g.at[jj])
        if jj >= 2:
          pltpu.make_async_copy(h_hbm.at[sd_g.at[jj].at[0]], rows[bc],
                                gsem[bc]).wait()
        do_scale(rows[bc])
        pltpu.async_copy(rows[bc], acc_s.at[sd_g.at[jj].at[1]], ssem[bc],
                         add=True)
      pltpu.make_async_copy(h_hbm.at[sd_g.at[G].at[0]], rows_a,
                            sem_g0).wait()
      pltpu.make_async_copy(h_hbm.at[sd_g.at[G + 1].at[0]], rows_b,
                            sem_g1).wait()
      grow = row0 + (gi + 1) * G
      pltpu.sync_copy(sd_hbm.at[pl.ds(grow, G + 2)], sd_g)
      return carry
    lax.fori_loop(0, ngroups, group, 0)

    pltpu.make_async_copy(rows_c, acc_s.at[sd_g.at[0].at[1]],
                          sem_s2).wait()
    plsc.subcore_barrier()

    pltpu.sync_copy(den_v, den_hbm.at[wid])
    pltpu.sync_copy(acc_s.at[pl.ds(s * stripe, stripe)],
                    acc_hbm.at[c].at[pl.ds(s * stripe, stripe)])

  mesh = plsc.VectorSubcoreMesh(core_axis_name="c", subcore_axis_name="s",
                                num_cores=NCS, num_subcores=NS)
  fn = pl.kernel(
      body,
      out_type=[jax.ShapeDtypeStruct((NCS, n_pad, d), jnp.float32),
                jax.ShapeDtypeStruct((NCS * NS, n_pad), jnp.float32)],
      mesh=mesh,
      scratch_types=[
          pltpu.VMEM((n_pad,), jnp.float32),
          pltpu.VMEM((n_pad,), jnp.float32),
          pltpu.VMEM((G_CHUNKS + 2, 2, K_EDGES), jnp.int32),
          pltpu.VMEM((K_EDGES,), jnp.float32),
          pltpu.VMEM((K_EDGES, d), jnp.float32),
          pltpu.VMEM((K_EDGES, d), jnp.float32),
          pltpu.VMEM((K_EDGES, d), jnp.float32),
          pltpu.VMEM_SHARED((n_pad, d), jnp.float32),
          pltpu.SemaphoreType.DMA,
          pltpu.SemaphoreType.DMA,
          pltpu.SemaphoreType.DMA,
          pltpu.SemaphoreType.DMA,
          pltpu.SemaphoreType.DMA,
          pltpu.SemaphoreType.DMA,
      ],
      compiler_params=pltpu.CompilerParams(needs_layout_passes=False),
  )
  return fn(h, sd, asad)


def kernel(x, edge_index, W1, a_src1, a_dst1, b1, W2, a_src2, a_dst2, b2,
           Wo, bo):
  n, d_in = x.shape
  d_h = W1.shape[1]
  n_classes = Wo.shape[1]
  e = edge_index.shape[1]
  n_pad = _round_up(n + 1, 128)
  e_pad = _round_up(e + n, NCS * NS * K_EDGES * G_CHUNKS)
  e_tot = e_pad + (G_CHUNKS + 2) * K_EDGES

  ei = edge_index.astype(jnp.int32)
  loop = jnp.arange(n, dtype=jnp.int32)
  pad_e = e_tot - e - n
  srcf = jnp.concatenate(
      [ei[0], loop, jnp.zeros((pad_e,), jnp.int32)]).reshape(-1, K_EDGES)
  dstf = jnp.concatenate(
      [ei[1], loop, jnp.full((pad_e,), n, jnp.int32)]).reshape(-1, K_EDGES)
  sd = jnp.stack([srcf, dstf], axis=1)

  x_pad = jnp.pad(x, ((0, n_pad - n), (0, 0)))

  vec = lambda a: a.reshape(1, -1)

  h1, asad1 = pl.pallas_call(
      _tc_in_body,
      out_shape=[jax.ShapeDtypeStruct((n_pad, d_h), jnp.float32),
                 jax.ShapeDtypeStruct((1, n_pad), jnp.float32)],
  )(x_pad, W1, vec(a_src1), vec(a_dst1))

  acc1, den1 = _sc_edge_call(n_pad, e_pad, d_h, h1, sd, asad1)

  h2, asad2 = pl.pallas_call(
      _tc_mid_body,
      out_shape=[jax.ShapeDtypeStruct((n_pad, d_h), jnp.float32),
                 jax.ShapeDtypeStruct((1, n_pad), jnp.float32)],
  )(acc1, den1, vec(b1), W2, vec(a_src2), vec(a_dst2))

  acc2, den2 = _sc_edge_call(n_pad, e_pad, d_h, h2, sd, asad2)

  out = pl.pallas_call(
      _tc_head_body,
      out_shape=jax.ShapeDtypeStruct((n_pad, n_classes), jnp.float32),
  )(acc2, den2, vec(b2), Wo, vec(bo))

  return out[:n]
